# baseline (speedup 1.0000x reference)
import jax
import jax.numpy as jnp
from jax import lax
from jax.experimental import pallas as pl
from jax.experimental.pallas import tpu as pltpu
from jax.experimental.pallas import tpu_sc as plsc

_NC = 2
_NS = 16
_NW = _NC * _NS
_L = 16

_BATCH = 16384
_BPW = _BATCH // _NW
_GCHUNK = 128
_NG = _BPW // _GCHUNK

_LN2 = 0.6931471805599453
_SQRT2 = 1.4142135623730951


def _vlog(x):
    bits = lax.bitcast_convert_type(x, jnp.int32)
    e = lax.shift_right_logical(bits, jnp.full((_L,), 23, jnp.int32)) - 127
    m_bits = lax.bitwise_or(
        lax.bitwise_and(bits, jnp.full((_L,), 0x007FFFFF, jnp.int32)),
        jnp.full((_L,), 0x3F800000, jnp.int32),
    )
    m = lax.bitcast_convert_type(m_bits, jnp.float32)
    big = m > _SQRT2
    m = jnp.where(big, m * 0.5, m)
    e = e + jnp.where(big, 1, 0)
    t = (m - 1.0) / (m + 1.0)
    t2 = t * t
    poly = 2.0 * t * (1.0 + t2 * (1.0 / 3.0 + t2 * (0.2 + t2 * (1.0 / 7.0 + t2 / 9.0))))
    return e.astype(jnp.float32) * _LN2 + poly


def _sc_body(s_hbm, a_hbm, p_hbm, ub_hbm, w_hbm, part_hbm, out_hbm,
             s_v, a_v, p_v, idx_v, vals_v, scal_v, acc_v, red_v, sem):
    cid = lax.axis_index("c")
    sid = lax.axis_index("s")
    wid = sid * _NC + cid
    base = wid * _BPW

    pltpu.sync_copy(s_hbm.at[pl.ds(base, _BPW)], s_v)
    pltpu.sync_copy(a_hbm.at[pl.ds(base, _BPW)], a_v)
    pltpu.sync_copy(p_hbm.at[pl.ds(base, _BPW)], p_v)
    pltpu.sync_copy(ub_hbm, scal_v)

    for j in range(_NG):
        for k in range(_GCHUNK // _L):
            i = j * (_GCHUNK // _L) + k
            sv = s_v[pl.ds(i * _L, _L)]
            av = a_v[pl.ds(i * _L, _L)]
            idx_v[j, pl.ds(k * _L, _L)] = av * 1000000 + sv

    copies = [
        pltpu.make_async_copy(w_hbm.at[idx_v.at[j]],
                              vals_v.at[pl.ds(j * _GCHUNK, _GCHUNK)], sem)
        for j in range(_NG)
    ]
    for c in copies:
        c.start()
    for c in copies:
        c.wait()

    ub = scal_v[pl.ds(0, _L)]
    dl = scal_v[pl.ds(_L, _L)]

    acc = jnp.zeros((_L,), jnp.float32)
    for i in range(_BPW // _L):
        g = vals_v[pl.ds(i * _L, _L)]
        pv = p_v[pl.ds(i * _L, _L)]
        acc = acc + pv * _vlog(ub - dl * g)
    acc_v[...] = acc

    pltpu.sync_copy(acc_v, part_hbm.at[cid, sid])
    plsc.subcore_barrier()

    @pl.when(sid == 0)
    def _():
        pltpu.sync_copy(part_hbm.at[cid], red_v)
        tot = jnp.zeros((_L,), jnp.float32)
        for w in range(_NS):
            tot = tot + red_v[w, pl.ds(0, _L)]
        total = tot[0]
        for l in range(1, _L):
            total = total + tot[l]
        acc_v[...] = jnp.full((_L,), -total, jnp.float32)
        pltpu.sync_copy(acc_v, out_hbm.at[cid])


@jax.jit
def _run(s, a, p, scal, w_flat):
    mesh = plsc.VectorSubcoreMesh(core_axis_name="c", subcore_axis_name="s")
    return pl.kernel(
        _sc_body,
        out_type=(jax.ShapeDtypeStruct((_NC, _NS, _L), jnp.float32),
                  jax.ShapeDtypeStruct((_NC, _L), jnp.float32)),
        mesh=mesh,
        scratch_types=[
            pltpu.VMEM((_BPW,), jnp.int32),
            pltpu.VMEM((_BPW,), jnp.int32),
            pltpu.VMEM((_BPW,), jnp.float32),
            pltpu.VMEM((_NG, _GCHUNK), jnp.int32),
            pltpu.VMEM((_BPW,), jnp.float32),
            pltpu.VMEM((2 * _L,), jnp.float32),
            pltpu.VMEM((_L,), jnp.float32),
            pltpu.VMEM((_NS, _L), jnp.float32),
            pltpu.SemaphoreType.DMA,
        ],
    )(s, a, p, scal, w_flat)


def kernel(s, a, p, upper_bound, delta, weights):
    scal = jnp.concatenate([
        jnp.full((_L,), upper_bound, jnp.float32),
        jnp.full((_L,), delta, jnp.float32),
    ])
    w_flat = weights.T.reshape(-1)
    _, out = _run(s, a, p, scal, w_flat)
    return out[0, 0] + out[1, 0]


# speedup vs baseline: 66.2859x; 66.2859x over previous
import jax
import jax.numpy as jnp
from jax import lax
from jax.experimental import pallas as pl
from jax.experimental.pallas import tpu as pltpu
from jax.experimental.pallas import tpu_sc as plsc

_NC = 2
_NS = 16
_NW = _NC * _NS
_L = 16

_BATCH = 16384
_BPW = _BATCH // _NW
_SLOTS = 64
_NCHUNK = _BPW // _SLOTS
_NGRP = _SLOTS // _L

_LN2 = 0.6931471805599453
_SQRT2 = 1.4142135623730951


def _vlog(x):
    bits = lax.bitcast_convert_type(x, jnp.int32)
    e = lax.shift_right_logical(bits, jnp.full((_L,), 23, jnp.int32)) - 127
    m_bits = lax.bitwise_or(
        lax.bitwise_and(bits, jnp.full((_L,), 0x007FFFFF, jnp.int32)),
        jnp.full((_L,), 0x3F800000, jnp.int32),
    )
    m = lax.bitcast_convert_type(m_bits, jnp.float32)
    big = m > _SQRT2
    m = jnp.where(big, m * 0.5, m)
    e = e + jnp.where(big, 1, 0)
    t = (m - 1.0) / (m + 1.0)
    t2 = t * t
    poly = 2.0 * t * (1.0 + t2 * (1.0 / 3.0 + t2 * (0.2 + t2 * (1.0 / 7.0 + t2 / 9.0))))
    return e.astype(jnp.float32) * _LN2 + poly


def _sc_body(s_hbm, a_hbm, p_hbm, ub_hbm, wt_hbm, part_hbm, out_hbm,
             s_v, a_v, p_v, land_v, scal_v, acc_v, red_v, sem):
    cid = lax.axis_index("c")
    sid = lax.axis_index("s")
    wid = sid * _NC + cid
    base = wid * _BPW

    pltpu.sync_copy(s_hbm.at[pl.ds(base, _BPW)], s_v)
    pltpu.sync_copy(a_hbm.at[pl.ds(base, _BPW)], a_v)
    pltpu.sync_copy(p_hbm.at[pl.ds(base, _BPW)], p_v)
    pltpu.sync_copy(ub_hbm, scal_v)

    ub = scal_v[pl.ds(0, _L)]
    dl = scal_v[pl.ds(_L, _L)]

    drain = pltpu.make_async_copy(wt_hbm.at[pl.ds(0, 8), pl.ds(0, 128)],
                                  land_v.at[0], sem)

    def round_body(r, acc):
        def issue(i, _):
            sv = s_v[pl.ds((r * _NGRP + i) * _L, _L)]
            av = a_v[pl.ds((r * _NGRP + i) * _L, _L)]
            for k in range(_L):
                sj = sv[k]
                aj = av[k]
                ab = pl.multiple_of(lax.shift_left(
                    lax.shift_right_logical(aj, 3), 3), 8)
                sb = pl.multiple_of(lax.shift_left(
                    lax.shift_right_logical(sj, 7), 7), 128)
                pltpu.make_async_copy(
                    wt_hbm.at[pl.ds(ab, 8), pl.ds(sb, 128)],
                    land_v.at[i * _L + k], sem).start()
            return ()

        lax.fori_loop(0, _NGRP, issue, ())

        def dr(i, _):
            drain.wait()
            return ()
        lax.fori_loop(0, _SLOTS, dr, ())

        def pick(i, acc):
            sv = s_v[pl.ds((r * _NGRP + i) * _L, _L)]
            av = a_v[pl.ds((r * _NGRP + i) * _L, _L)]
            packed = jnp.zeros((_L,), jnp.float32)
            for k in range(_L):
                sj = sv[k]
                ar = lax.bitwise_and(av[k], 7)
                cb = lax.shift_left(lax.shift_right_logical(
                    lax.bitwise_and(sj, 127), 4), 4)
                blk = land_v[i * _L + k, ar, pl.ds(cb, _L)]
                rk = lax.bitwise_and(sj, 15)
                val = blk[0]
                for l in range(1, _L):
                    val = jnp.where(rk == l, blk[l], val)
                packed = jnp.where(lax.iota(jnp.int32, _L) == k,
                                   jnp.full((_L,), val, jnp.float32),
                                   packed)
            pv = p_v[pl.ds((r * _NGRP + i) * _L, _L)]
            return acc + pv * _vlog(ub - dl * packed)

        return lax.fori_loop(0, _NGRP, pick, acc)

    acc = lax.fori_loop(0, _NCHUNK, round_body, jnp.zeros((_L,), jnp.float32))
    acc_v[...] = acc

    pltpu.sync_copy(acc_v, part_hbm.at[cid, sid])
    plsc.subcore_barrier()

    @pl.when(sid == 0)
    def _():
        pltpu.sync_copy(part_hbm.at[cid], red_v)
        tot = jnp.zeros((_L,), jnp.float32)
        for w in range(_NS):
            tot = tot + red_v[w, pl.ds(0, _L)]
        total = tot[0]
        for l in range(1, _L):
            total = total + tot[l]
        acc_v[...] = jnp.full((_L,), -total, jnp.float32)
        pltpu.sync_copy(acc_v, out_hbm.at[cid])


@jax.jit
def _run(s, a, p, scal, wt):
    mesh = plsc.VectorSubcoreMesh(core_axis_name="c", subcore_axis_name="s")
    return pl.kernel(
        _sc_body,
        out_type=(jax.ShapeDtypeStruct((_NC, _NS, _L), jnp.float32),
                  jax.ShapeDtypeStruct((_NC, _L), jnp.float32)),
        mesh=mesh,
        scratch_types=[
            pltpu.VMEM((_BPW,), jnp.int32),
            pltpu.VMEM((_BPW,), jnp.int32),
            pltpu.VMEM((_BPW,), jnp.float32),
            pltpu.VMEM((_SLOTS, 8, 128), jnp.float32),
            pltpu.VMEM((2 * _L,), jnp.float32),
            pltpu.VMEM((_L,), jnp.float32),
            pltpu.VMEM((_NS, _L), jnp.float32),
            pltpu.SemaphoreType.DMA,
        ],
    )(s, a, p, scal, wt)


def kernel(s, a, p, upper_bound, delta, weights):
    scal = jnp.concatenate([
        jnp.full((_L,), upper_bound, jnp.float32),
        jnp.full((_L,), delta, jnp.float32),
    ])
    _, out = _run(s, a, p, scal, weights.T)
    return out[0, 0] + out[1, 0]


# speedup vs baseline: 68.8593x; 1.0388x over previous
import jax
import jax.numpy as jnp
from jax import lax
from jax.experimental import pallas as pl
from jax.experimental.pallas import tpu as pltpu
from jax.experimental.pallas import tpu_sc as plsc

_NC = 2
_NS = 16
_NW = _NC * _NS
_L = 16

_BATCH = 16384
_BPW = _BATCH // _NW
_SLOTS = 32
_NCHUNK = _BPW // _SLOTS
_NGRP = _SLOTS // _L

_LN2 = 0.6931471805599453
_SQRT2 = 1.4142135623730951


def _vlog(x):
    bits = lax.bitcast_convert_type(x, jnp.int32)
    e = lax.shift_right_logical(bits, jnp.full((_L,), 23, jnp.int32)) - 127
    m_bits = lax.bitwise_or(
        lax.bitwise_and(bits, jnp.full((_L,), 0x007FFFFF, jnp.int32)),
        jnp.full((_L,), 0x3F800000, jnp.int32),
    )
    m = lax.bitcast_convert_type(m_bits, jnp.float32)
    big = m > _SQRT2
    m = jnp.where(big, m * 0.5, m)
    e = e + jnp.where(big, 1, 0)
    t = (m - 1.0) / (m + 1.0)
    t2 = t * t
    poly = 2.0 * t * (1.0 + t2 * (1.0 / 3.0 + t2 * (0.2 + t2 * (1.0 / 7.0 + t2 / 9.0))))
    return e.astype(jnp.float32) * _LN2 + poly


def _sc_body(s_hbm, a_hbm, p_hbm, ub_hbm, wt_hbm, part_hbm, out_hbm,
             s_v, a_v, p_v, land_v, scal_v, acc_v, red_v, sem0, sem1):
    cid = lax.axis_index("c")
    sid = lax.axis_index("s")
    wid = sid * _NC + cid
    base = wid * _BPW

    pltpu.sync_copy(s_hbm.at[pl.ds(base, _BPW)], s_v)
    pltpu.sync_copy(a_hbm.at[pl.ds(base, _BPW)], a_v)
    pltpu.sync_copy(p_hbm.at[pl.ds(base, _BPW)], p_v)
    pltpu.sync_copy(ub_hbm, scal_v)

    ub = scal_v[pl.ds(0, _L)]
    dl = scal_v[pl.ds(_L, _L)]

    sems = (sem0, sem1)
    drains = tuple(
        pltpu.make_async_copy(wt_hbm.at[pl.ds(0, 8), pl.ds(0, 128)],
                              land_v.at[par, 0], sems[par])
        for par in range(2))

    def issue(c, par):
        def grp(i, _):
            sv = s_v[pl.ds(c * _SLOTS + i * _L, _L)]
            av = a_v[pl.ds(c * _SLOTS + i * _L, _L)]
            for k in range(_L):
                sj = sv[k]
                aj = av[k]
                ab = pl.multiple_of(lax.shift_left(
                    lax.shift_right_logical(aj, 3), 3), 8)
                sb = pl.multiple_of(lax.shift_left(
                    lax.shift_right_logical(sj, 7), 7), 128)
                pltpu.make_async_copy(
                    wt_hbm.at[pl.ds(ab, 8), pl.ds(sb, 128)],
                    land_v.at[par, i * _L + k], sems[par]).start()
            return ()
        lax.fori_loop(0, _NGRP, grp, ())

    def drain(par):
        def dr(i, _):
            drains[par].wait()
            return ()
        lax.fori_loop(0, _SLOTS, dr, ())

    def pick(c, par, acc):
        def grp(i, acc):
            sv = s_v[pl.ds(c * _SLOTS + i * _L, _L)]
            av = a_v[pl.ds(c * _SLOTS + i * _L, _L)]
            packed = jnp.zeros((_L,), jnp.float32)
            for k in range(_L):
                sj = sv[k]
                ar = lax.bitwise_and(av[k], 7)
                cb = lax.shift_left(lax.shift_right_logical(
                    lax.bitwise_and(sj, 127), 4), 4)
                blk = land_v[par, i * _L + k, ar, pl.ds(cb, _L)]
                rk = lax.bitwise_and(sj, 15)
                val = blk[0]
                for l in range(1, _L):
                    val = jnp.where(rk == l, blk[l], val)
                packed = jnp.where(lax.iota(jnp.int32, _L) == k,
                                   jnp.full((_L,), val, jnp.float32),
                                   packed)
            pv = p_v[pl.ds(c * _SLOTS + i * _L, _L)]
            return acc + pv * _vlog(ub - dl * packed)
        return lax.fori_loop(0, _NGRP, grp, acc)

    issue(0, 0)

    def round_body(r, acc):
        issue(2 * r + 1, 1)
        drain(0)
        acc = pick(2 * r, 0, acc)

        @pl.when(r < _NCHUNK // 2 - 1)
        def _():
            issue(2 * r + 2, 0)
        drain(1)
        return pick(2 * r + 1, 1, acc)

    acc = lax.fori_loop(0, _NCHUNK // 2, round_body,
                        jnp.zeros((_L,), jnp.float32))
    acc_v[...] = acc

    pltpu.sync_copy(acc_v, part_hbm.at[cid, sid])
    plsc.subcore_barrier()

    @pl.when(sid == 0)
    def _():
        pltpu.sync_copy(part_hbm.at[cid], red_v)
        tot = jnp.zeros((_L,), jnp.float32)
        for w in range(_NS):
            tot = tot + red_v[w, pl.ds(0, _L)]
        total = tot[0]
        for l in range(1, _L):
            total = total + tot[l]
        acc_v[...] = jnp.full((_L,), -total, jnp.float32)
        pltpu.sync_copy(acc_v, out_hbm.at[cid])


@jax.jit
def _run(s, a, p, scal, wt):
    mesh = plsc.VectorSubcoreMesh(core_axis_name="c", subcore_axis_name="s")
    return pl.kernel(
        _sc_body,
        out_type=(jax.ShapeDtypeStruct((_NC, _NS, _L), jnp.float32),
                  jax.ShapeDtypeStruct((_NC, _L), jnp.float32)),
        mesh=mesh,
        scratch_types=[
            pltpu.VMEM((_BPW,), jnp.int32),
            pltpu.VMEM((_BPW,), jnp.int32),
            pltpu.VMEM((_BPW,), jnp.float32),
            pltpu.VMEM((2, _SLOTS, 8, 128), jnp.float32),
            pltpu.VMEM((2 * _L,), jnp.float32),
            pltpu.VMEM((_L,), jnp.float32),
            pltpu.VMEM((_NS, _L), jnp.float32),
            pltpu.SemaphoreType.DMA,
            pltpu.SemaphoreType.DMA,
        ],
    )(s, a, p, scal, wt)


def kernel(s, a, p, upper_bound, delta, weights):
    scal = jnp.concatenate([
        jnp.full((_L,), upper_bound, jnp.float32),
        jnp.full((_L,), delta, jnp.float32),
    ])
    _, out = _run(s, a, p, scal, weights.T)
    return out[0, 0] + out[1, 0]


# speedup vs baseline: 93.2249x; 1.3538x over previous
import jax
import jax.numpy as jnp
from jax import lax
from jax.experimental import pallas as pl
from jax.experimental.pallas import tpu as pltpu
from jax.experimental.pallas import tpu_sc as plsc

_NC = 2
_NS = 16
_NW = _NC * _NS
_L = 16

_BATCH = 16384
_BPW = _BATCH // _NW
_SLOTS = 32
_NCHUNK = _BPW // _SLOTS
_NGRP = _SLOTS // _L

_LN2 = 0.6931471805599453
_SQRT2 = 1.4142135623730951


def _vlog(x):
    bits = lax.bitcast_convert_type(x, jnp.int32)
    e = lax.shift_right_logical(bits, jnp.full((_L,), 23, jnp.int32)) - 127
    m_bits = lax.bitwise_or(
        lax.bitwise_and(bits, jnp.full((_L,), 0x007FFFFF, jnp.int32)),
        jnp.full((_L,), 0x3F800000, jnp.int32),
    )
    m = lax.bitcast_convert_type(m_bits, jnp.float32)
    big = m > _SQRT2
    m = jnp.where(big, m * 0.5, m)
    e = e + jnp.where(big, 1, 0)
    t = (m - 1.0) / (m + 1.0)
    t2 = t * t
    poly = 2.0 * t * (1.0 + t2 * (1.0 / 3.0 + t2 * (0.2 + t2 * (1.0 / 7.0 + t2 / 9.0))))
    return e.astype(jnp.float32) * _LN2 + poly


def _sc_body(s_hbm, a_hbm, p_hbm, ub_hbm, wt_hbm, part_hbm, out_hbm,
             s_v, a_v, p_v, land_v, scal_v, acc_v, red_v, sem0, sem1):
    cid = lax.axis_index("c")
    sid = lax.axis_index("s")
    wid = sid * _NC + cid
    base = wid * _BPW

    pltpu.sync_copy(s_hbm.at[pl.ds(base, _BPW)], s_v)
    pltpu.sync_copy(a_hbm.at[pl.ds(base, _BPW)], a_v)
    pltpu.sync_copy(p_hbm.at[pl.ds(base, _BPW)], p_v)
    pltpu.sync_copy(ub_hbm, scal_v)

    ub = scal_v[pl.ds(0, _L)]
    dl = scal_v[pl.ds(_L, _L)]

    sems = (sem0, sem1)
    drains = tuple(
        pltpu.make_async_copy(wt_hbm.at[pl.ds(0, 8), pl.ds(0, 128)],
                              land_v.at[par, 0], sems[par])
        for par in range(2))

    def issue(c, par):
        def grp(i, _):
            sv = s_v[pl.ds(c * _SLOTS + i * _L, _L)]
            av = a_v[pl.ds(c * _SLOTS + i * _L, _L)]
            for k in range(_L):
                sj = sv[k]
                aj = av[k]
                ab = pl.multiple_of(lax.shift_left(
                    lax.shift_right_logical(aj, 3), 3), 8)
                sb = pl.multiple_of(lax.shift_left(
                    lax.shift_right_logical(sj, 7), 7), 128)
                pltpu.make_async_copy(
                    wt_hbm.at[pl.ds(ab, 8), pl.ds(sb, 128)],
                    land_v.at[par, i * _L + k], sems[par]).start()
            return ()
        lax.fori_loop(0, _NGRP, grp, ())

    def drain(par):
        def dr(i, _):
            drains[par].wait()
            return ()
        lax.fori_loop(0, _SLOTS, dr, ())

    def pick(c, par, acc):
        lane = lax.iota(jnp.int32, _L)
        parv = jnp.full((_L,), par, jnp.int32)
        def grp(i, acc):
            sv = s_v[pl.ds(c * _SLOTS + i * _L, _L)]
            av = a_v[pl.ds(c * _SLOTS + i * _L, _L)]
            slot = lane + i * _L
            row = lax.bitwise_and(av, jnp.full((_L,), 7, jnp.int32))
            col = lax.bitwise_and(sv, jnp.full((_L,), 127, jnp.int32))
            packed = plsc.load_gather(land_v, [parv, slot, row, col])
            pv = p_v[pl.ds(c * _SLOTS + i * _L, _L)]
            return acc + pv * _vlog(ub - dl * packed)
        return lax.fori_loop(0, _NGRP, grp, acc)

    issue(0, 0)

    def round_body(r, acc):
        issue(2 * r + 1, 1)
        drain(0)
        acc = pick(2 * r, 0, acc)

        @pl.when(r < _NCHUNK // 2 - 1)
        def _():
            issue(2 * r + 2, 0)
        drain(1)
        return pick(2 * r + 1, 1, acc)

    acc = lax.fori_loop(0, _NCHUNK // 2, round_body,
                        jnp.zeros((_L,), jnp.float32))
    acc_v[...] = acc

    pltpu.sync_copy(acc_v, part_hbm.at[cid, sid])
    plsc.subcore_barrier()

    @pl.when(sid == 0)
    def _():
        pltpu.sync_copy(part_hbm.at[cid], red_v)
        tot = jnp.zeros((_L,), jnp.float32)
        for w in range(_NS):
            tot = tot + red_v[w, pl.ds(0, _L)]
        total = tot[0]
        for l in range(1, _L):
            total = total + tot[l]
        acc_v[...] = jnp.full((_L,), -total, jnp.float32)
        pltpu.sync_copy(acc_v, out_hbm.at[cid])


@jax.jit
def _run(s, a, p, scal, wt):
    mesh = plsc.VectorSubcoreMesh(core_axis_name="c", subcore_axis_name="s")
    return pl.kernel(
        _sc_body,
        out_type=(jax.ShapeDtypeStruct((_NC, _NS, _L), jnp.float32),
                  jax.ShapeDtypeStruct((_NC, _L), jnp.float32)),
        mesh=mesh,
        compiler_params=pltpu.CompilerParams(needs_layout_passes=False),
        scratch_types=[
            pltpu.VMEM((_BPW,), jnp.int32),
            pltpu.VMEM((_BPW,), jnp.int32),
            pltpu.VMEM((_BPW,), jnp.float32),
            pltpu.VMEM((2, _SLOTS, 8, 128), jnp.float32),
            pltpu.VMEM((2 * _L,), jnp.float32),
            pltpu.VMEM((_L,), jnp.float32),
            pltpu.VMEM((_NS, _L), jnp.float32),
            pltpu.SemaphoreType.DMA,
            pltpu.SemaphoreType.DMA,
        ],
    )(s, a, p, scal, wt)


def kernel(s, a, p, upper_bound, delta, weights):
    scal = jnp.concatenate([
        jnp.full((_L,), upper_bound, jnp.float32),
        jnp.full((_L,), delta, jnp.float32),
    ])
    _, out = _run(s, a, p, scal, weights.T)
    return out[0, 0] + out[1, 0]
